# Initial kernel scaffold; baseline (speedup 1.0000x reference)
import jax
import jax.numpy as jnp
from jax.experimental import pallas as pl


def kernel(x, embeddings_table):
    raise NotImplementedError("write your pallas kernel here")


# speedup vs baseline: 8.1708x; 8.1708x over previous
import jax
import jax.numpy as jnp
from jax import lax
from jax.experimental import pallas as pl
from jax.experimental.pallas import tpu as pltpu
from jax.experimental.pallas import tpu_sc as plsc

NUM_UNITS = 32
MAX_REL = 128
LQ = 2048
LK = 2048
E_ROWS = LQ + LK - 1
BAND_LO = LK - 1 - MAX_REL
BAND_ROWS = 2 * MAX_REL + 1
NW = 32
ROWS_PER_W = LQ // NW
ROW_W = LK * NUM_UNITS


def _sc_body(table_hbm, out_hbm, e_vmem):
    c = lax.axis_index("c")
    s = lax.axis_index("s")
    wid = s * 2 + c

    pltpu.sync_copy(table_hbm, e_vmem.at[pl.ds(BAND_LO * NUM_UNITS, BAND_ROWS * NUM_UNITS)])

    t0a = e_vmem[pl.ds(BAND_LO * NUM_UNITS, 16)]
    t0b = e_vmem[pl.ds(BAND_LO * NUM_UNITS + 16, 16)]
    t1a = e_vmem[pl.ds((BAND_LO + BAND_ROWS - 1) * NUM_UNITS, 16)]
    t1b = e_vmem[pl.ds((BAND_LO + BAND_ROWS - 1) * NUM_UNITS + 16, 16)]

    def fill_head(k, _):
        e_vmem[pl.ds(k * NUM_UNITS, 16)] = t0a
        e_vmem[pl.ds(k * NUM_UNITS + 16, 16)] = t0b
        return 0

    lax.fori_loop(0, BAND_LO, fill_head, 0)

    def fill_tail(k, _):
        e_vmem[pl.ds(k * NUM_UNITS, 16)] = t1a
        e_vmem[pl.ds(k * NUM_UNITS + 16, 16)] = t1b
        return 0

    lax.fori_loop(BAND_LO + BAND_ROWS, E_ROWS, fill_tail, 0)

    def row_step(r, _):
        i = wid * ROWS_PER_W + r
        pltpu.sync_copy(e_vmem.at[pl.ds((LK - 1 - i) * NUM_UNITS, ROW_W)],
                        out_hbm.at[pl.ds(i * ROW_W, ROW_W)])
        return 0

    lax.fori_loop(0, ROWS_PER_W, row_step, 0)


def kernel(x, embeddings_table):
    del x
    run = pl.kernel(
        _sc_body,
        out_type=jax.ShapeDtypeStruct((LQ * ROW_W,), jnp.float32),
        mesh=plsc.VectorSubcoreMesh(core_axis_name="c", subcore_axis_name="s"),
        scratch_types=[
            pltpu.VMEM((E_ROWS * NUM_UNITS,), jnp.float32),
        ],
    )
    out = run(embeddings_table.reshape(-1))
    return out.reshape(LQ, LK, NUM_UNITS)
